# baseline (speedup 1.0000x reference)
import jax
import jax.numpy as jnp
from jax import lax
from jax.experimental import pallas as pl

E = 320000
D = 128
R = 512
B = 512


def _fused_kernel(idx_ref, table_ref, n1_ref, n2_ref, hn_ref, tn_ref,
                  pos_ref, nh_ref, nt_ref, r_ref):
    idx = idx_ref[...]
    iota = lax.broadcasted_iota(jnp.int32, (B, R), 1)
    onehot = (iota == idx[:, None]).astype(jnp.float32)
    r = jnp.dot(onehot, table_ref[...], preferred_element_type=jnp.float32)
    r_ref[...] = r

    n1 = n1_ref[...]
    n2 = n2_ref[...]
    pos_ref[...] = jnp.sum(jnp.abs(n1 * r - n2), axis=1)
    nh_ref[...] = jnp.sum(jnp.abs(hn_ref[...] * r - n2), axis=1)
    nt_ref[...] = jnp.sum(jnp.abs(n1 * r - tn_ref[...]), axis=1)


def kernel(update_rel_embed, edge_type, node1_encoder_result,
           node2_encoder_result, head_neg_encoder_result,
           tail_neg_encoder_result):
    idx = edge_type.astype(jnp.int32)
    grid = (E // B,)
    edge_spec = pl.BlockSpec((B, D), lambda i: (i, 0))
    score_spec = pl.BlockSpec((B,), lambda i: (i,))

    pos, nh, nt, r = pl.pallas_call(
        _fused_kernel,
        grid=grid,
        in_specs=[
            score_spec,
            pl.BlockSpec((R, D), lambda i: (0, 0)),
            edge_spec, edge_spec, edge_spec, edge_spec,
        ],
        out_specs=[score_spec, score_spec, score_spec, edge_spec],
        out_shape=[
            jax.ShapeDtypeStruct((E,), jnp.float32),
            jax.ShapeDtypeStruct((E,), jnp.float32),
            jax.ShapeDtypeStruct((E,), jnp.float32),
            jax.ShapeDtypeStruct((E, D), jnp.float32),
        ],
    )(idx, update_rel_embed, node1_encoder_result, node2_encoder_result,
      head_neg_encoder_result, tail_neg_encoder_result)

    return (pos, nh, nt, r)


# speedup vs baseline: 1.0437x; 1.0437x over previous
import jax
import jax.numpy as jnp
from jax import lax
from jax.experimental import pallas as pl

E = 320000
D = 128
R = 512
B = 5000


def _fused_kernel(idx_ref, table_ref, n1_ref, n2_ref, hn_ref, tn_ref,
                  pos_ref, nh_ref, nt_ref, r_ref):
    idx = idx_ref[:, 0]
    iota = lax.broadcasted_iota(jnp.int32, (B, R), 1)
    onehot = (iota == idx[:, None]).astype(jnp.float32)
    r = jnp.dot(onehot, table_ref[...], preferred_element_type=jnp.float32)
    r_ref[...] = r

    n1 = n1_ref[...]
    n2 = n2_ref[...]
    pos_ref[:, 0] = jnp.sum(jnp.abs(n1 * r - n2), axis=1)
    nh_ref[:, 0] = jnp.sum(jnp.abs(hn_ref[...] * r - n2), axis=1)
    nt_ref[:, 0] = jnp.sum(jnp.abs(n1 * r - tn_ref[...]), axis=1)


def kernel(update_rel_embed, edge_type, node1_encoder_result,
           node2_encoder_result, head_neg_encoder_result,
           tail_neg_encoder_result):
    idx2d = edge_type.astype(jnp.int32).reshape(E, 1)
    grid = (E // B,)
    edge_spec = pl.BlockSpec((B, D), lambda i: (i, 0))
    score_spec = pl.BlockSpec((B, 1), lambda i: (i, 0))

    pos, nh, nt, r = pl.pallas_call(
        _fused_kernel,
        grid=grid,
        in_specs=[
            pl.BlockSpec((B, 1), lambda i: (i, 0)),
            pl.BlockSpec((R, D), lambda i: (0, 0)),
            edge_spec, edge_spec, edge_spec, edge_spec,
        ],
        out_specs=[score_spec, score_spec, score_spec, edge_spec],
        out_shape=[
            jax.ShapeDtypeStruct((E, 1), jnp.float32),
            jax.ShapeDtypeStruct((E, 1), jnp.float32),
            jax.ShapeDtypeStruct((E, 1), jnp.float32),
            jax.ShapeDtypeStruct((E, D), jnp.float32),
        ],
    )(idx2d, update_rel_embed, node1_encoder_result, node2_encoder_result,
      head_neg_encoder_result, tail_neg_encoder_result)

    return (pos.reshape(E), nh.reshape(E), nt.reshape(E), r)
